# Initial kernel scaffold; baseline (speedup 1.0000x reference)
import jax
import jax.numpy as jnp
from jax.experimental import pallas as pl


def kernel(x_real, x_imag, edge_index, Wr, br, Wi, bi):
    raise NotImplementedError("write your pallas kernel here")


# speedup vs baseline: 1.1912x; 1.1912x over previous
import jax
import jax.numpy as jnp
from jax.experimental import pallas as pl
from jax.experimental.pallas import tpu as pltpu

N = 10000
D = 128
K = 3
EXP = -0.25
ALPHA = 0.5

_ROW_BLK = 2000


def _dense_body(yr_ref, ur_ref, ui_ref, wr_ref, br_ref, wi_ref, bi_ref,
                or_ref, oi_ref):
    wr_eff = wr_ref[0] + 0.5 * wr_ref[1] + 0.25 * wr_ref[2]
    wi_eff = wi_ref[0] + 0.5 * wi_ref[1] + 0.25 * wi_ref[2]
    br_eff = br_ref[0] + 0.5 * br_ref[1] + 0.25 * br_ref[2]
    bi_eff = bi_ref[0] + 0.5 * bi_ref[1] + 0.25 * bi_ref[2]

    def mm(x, w):
        return jax.lax.dot_general(
            x, w, (((1,), (1,)), ((), ())),
            preferred_element_type=jnp.float32)

    yr = yr_ref[...]
    ur = ur_ref[...]
    ui = ui_ref[...]
    or_ref[...] = mm(ur, wr_eff) - mm(ui, wi_eff) + (br_eff - bi_eff)[None, :]
    oi_ref[...] = mm(yr, wi_eff) + mm(ui, wr_eff) + (br_eff + bi_eff)[None, :]


def _dense_pallas(y_real, u_real, u_imag, Wr, br, Wi, bi):
    grid = (N // _ROW_BLK,)
    row_spec = pl.BlockSpec((_ROW_BLK, D), lambda i: (i, 0))
    w_spec = pl.BlockSpec((K, D, D), lambda i: (0, 0, 0))
    b_spec = pl.BlockSpec((K, D), lambda i: (0, 0))
    return pl.pallas_call(
        _dense_body,
        grid=grid,
        in_specs=[row_spec, row_spec, row_spec, w_spec, b_spec, w_spec, b_spec],
        out_specs=[row_spec, row_spec],
        out_shape=[jax.ShapeDtypeStruct((N, D), jnp.float32),
                   jax.ShapeDtypeStruct((N, D), jnp.float32)],
    )(y_real, u_real, u_imag, Wr, br, Wi, bi)


def kernel(x_real, x_imag, edge_index, Wr, br, Wi, bi):
    row, col = edge_index[0], edge_index[1]
    ones = jnp.ones(row.shape[0], dtype=jnp.float32)
    d_out = jnp.zeros(N, jnp.float32).at[row].add(ones)
    d_in = jnp.zeros(N, jnp.float32).at[col].add(ones)
    a = jnp.where(d_out > 0, d_out ** EXP, 0.0)
    b_ = jnp.where(d_in > 0, d_in ** EXP, 0.0)
    w = a[row] * b_[col]

    X = jnp.concatenate([x_real, x_imag], axis=1)
    Sf = jnp.zeros((N, 2 * D), jnp.float32).at[row].add(w[:, None] * X[col])
    Sb = jnp.zeros((N, 2 * D), jnp.float32).at[col].add(w[:, None] * X[row])
    y_real = Sf[:, :D]
    u_real = ALPHA * y_real + (1.0 - ALPHA) * Sb[:, :D]
    u_imag = ALPHA * Sf[:, D:] + (1.0 - ALPHA) * Sb[:, D:]

    out_real, out_imag = _dense_pallas(y_real, u_real, u_imag, Wr, br, Wi, bi)
    return out_real, out_imag


# speedup vs baseline: 5.7863x; 4.8577x over previous
import functools

import jax
import jax.numpy as jnp
from jax import lax
from jax.experimental import pallas as pl
from jax.experimental.pallas import tpu as pltpu
from jax.experimental.pallas import tpu_sc as plsc

N = 10000
D = 128
K = 3
ALPHA = 0.5

NW = 32
FPW = 2
NPASS = D // (NW * FPW)
C = 1280
ROT = 7

_ROW_BLK = 2000

_MAGIC = 0x5F3759DF
_MASKHI = -65536


def _rsqrt_nr(x):
    u = plsc.bitcast(x, jnp.int32)
    u = jnp.int32(_MAGIC) - lax.shift_right_logical(u, jnp.int32(1))
    r = plsc.bitcast(u, jnp.float32)
    for _ in range(3):
        r = r * (1.5 - 0.5 * x * r * r)
    return r


def _sc_spmm(row, col, xp):
    E = row.shape[0]
    assert E % C == 0
    nchunk = E // C
    npair = nchunk // 2
    mesh = plsc.VectorSubcoreMesh(core_axis_name="c", subcore_axis_name="s")
    out_t = jax.ShapeDtypeStruct((NPASS * NW, 2 * N), jnp.float32)

    @functools.partial(
        pl.kernel, mesh=mesh,
        out_type=[out_t, out_t, out_t],
        compiler_params=pltpu.CompilerParams(needs_layout_passes=False),
        scratch_types=[
            pltpu.VMEM((N,), jnp.float32),
            pltpu.VMEM((N,), jnp.float32),
            pltpu.VMEM((2 * N,), jnp.int32),
            pltpu.VMEM((2 * N,), jnp.float32),
            pltpu.VMEM((2 * N,), jnp.float32),
            pltpu.VMEM((2 * N,), jnp.float32),
            pltpu.VMEM((C,), jnp.int32),
            pltpu.VMEM((C,), jnp.int32),
            pltpu.VMEM((C,), jnp.int32),
            pltpu.VMEM((C,), jnp.int32),
            pltpu.SemaphoreType.DMA,
            pltpu.SemaphoreType.DMA,
            pltpu.SemaphoreType.DMA,
            pltpu.SemaphoreType.DMA,
        ],
    )
    def k(row_h, col_h, xp_h, yr_h, ur_h, ui_h,
          da, db, xv, yr, yrt, ui, rb0, rb1, cb0, cb1, sr0, sr1, sc0, sc1):
        wid = lax.axis_index("s") * 2 + lax.axis_index("c")
        rbufs, cbufs = (rb0, rb1), (cb0, cb1)
        srs, scs = (sr0, sr1), (sc0, sc1)

        def chunk_of(c):
            return lax.rem(c + wid * ROT, nchunk)

        def start_chunk(ci, par):
            off = chunk_of(ci) * C
            pltpu.make_async_copy(
                row_h.at[pl.ds(off, C)], rbufs[par], srs[par]).start()
            pltpu.make_async_copy(
                col_h.at[pl.ds(off, C)], cbufs[par], scs[par]).start()

        def wait_chunk(par):
            pltpu.make_async_copy(
                row_h.at[pl.ds(0, C)], rbufs[par], srs[par]).wait()
            pltpu.make_async_copy(
                col_h.at[pl.ds(0, C)], cbufs[par], scs[par]).wait()

        def edge_pass(process_group):
            def pair_body(j, _):
                for par in (0, 1):
                    c = j * 2 + par
                    start_chunk(c + 1, 1 - par)
                    wait_chunk(par)

                    def g_body(g, _):
                        i0 = g * 16
                        rows = rbufs[par][pl.ds(i0, 16)]
                        cols = cbufs[par][pl.ds(i0, 16)]
                        process_group(rows, cols)
                        return 0

                    lax.fori_loop(0, C // 16, g_body, 0)
                return 0
            lax.fori_loop(0, npair, pair_body, 0)

        def zero_ref(ref, n):
            z = jnp.zeros((16,), ref.dtype)

            def b(i, _):
                ref[pl.ds(i * 16, 16)] = z
                return 0
            lax.fori_loop(0, n // 16, b, 0)

        zero_ref(da, N)
        zero_ref(db, N)
        start_chunk(0, 0)
        ones = jnp.ones((16,), jnp.float32)

        def deg_group(rows, cols):
            plsc.addupdate_scatter(da, [rows], ones)
            plsc.addupdate_scatter(db, [cols], ones)

        edge_pass(deg_group)

        def finalize(ref):
            def b(i, _):
                s = pl.ds(i * 16, 16)
                d = ref[s]
                r1 = _rsqrt_nr(d)
                val = r1 * _rsqrt_nr(r1)
                ref[s] = jnp.where(d > 0.0, val, 0.0)
                return 0
            lax.fori_loop(0, N // 16, b, 0)

        finalize(da)
        finalize(db)

        for p in range(NPASS):
            b_idx = wid + NW * p
            pltpu.sync_copy(xp_h.at[b_idx], xv)
            zero_ref(yr, 2 * N)
            zero_ref(yrt, 2 * N)
            zero_ref(ui, 2 * N)

            def acc_group(rows, cols):
                av = plsc.load_gather(da, [rows])
                bv = plsc.load_gather(db, [cols])
                w = av * bv
                hw = w * ALPHA
                r2 = rows * 2
                c2 = cols * 2
                for f in range(FPW):
                    rf = r2 + f if f else r2
                    cf = c2 + f if f else c2
                    pc = plsc.load_gather(xv, [cf])
                    pr = plsc.load_gather(xv, [rf])
                    mhi = jnp.int32(_MASKHI)
                    xr_c = plsc.bitcast(pc & mhi, jnp.float32)
                    xi_c = plsc.bitcast(
                        lax.shift_left(pc, jnp.int32(16)), jnp.float32)
                    xr_r = plsc.bitcast(pr & mhi, jnp.float32)
                    xi_r = plsc.bitcast(
                        lax.shift_left(pr, jnp.int32(16)), jnp.float32)
                    plsc.addupdate_scatter(yr, [rf], w * xr_c)
                    plsc.addupdate_scatter(yrt, [cf], w * xr_r)
                    plsc.addupdate_scatter(ui, [rf], hw * xi_c)
                    plsc.addupdate_scatter(ui, [cf], hw * xi_r)

            edge_pass(acc_group)

            def ucomb(i, _):
                s = pl.ds(i * 16, 16)
                yrt[s] = ALPHA * yr[s] + (1.0 - ALPHA) * yrt[s]
                return 0
            lax.fori_loop(0, 2 * N // 16, ucomb, 0)

            pltpu.sync_copy(yr, yr_h.at[b_idx])
            pltpu.sync_copy(yrt, ur_h.at[b_idx])
            pltpu.sync_copy(ui, ui_h.at[b_idx])

        wait_chunk(0)

    return k(row, col, xp)


def _dense_body(yr_ref, ur_ref, ui_ref, wr_ref, br_ref, wi_ref, bi_ref,
                or_ref, oi_ref):
    wr_eff = wr_ref[0] + 0.5 * wr_ref[1] + 0.25 * wr_ref[2]
    wi_eff = wi_ref[0] + 0.5 * wi_ref[1] + 0.25 * wi_ref[2]
    br_eff = br_ref[0] + 0.5 * br_ref[1] + 0.25 * br_ref[2]
    bi_eff = bi_ref[0] + 0.5 * bi_ref[1] + 0.25 * bi_ref[2]

    def mm(x, w):
        return jax.lax.dot_general(
            x, w, (((1,), (1,)), ((), ())),
            preferred_element_type=jnp.float32)

    yr = yr_ref[...]
    ur = ur_ref[...]
    ui = ui_ref[...]
    or_ref[...] = mm(ur, wr_eff) - mm(ui, wi_eff) + (br_eff - bi_eff)[None, :]
    oi_ref[...] = mm(yr, wi_eff) + mm(ui, wr_eff) + (br_eff + bi_eff)[None, :]


def _dense_pallas(y_real, u_real, u_imag, Wr, br, Wi, bi):
    grid = (N // _ROW_BLK,)
    row_spec = pl.BlockSpec((_ROW_BLK, D), lambda i: (i, 0))
    w_spec = pl.BlockSpec((K, D, D), lambda i: (0, 0, 0))
    b_spec = pl.BlockSpec((K, D), lambda i: (0, 0))
    return pl.pallas_call(
        _dense_body,
        grid=grid,
        in_specs=[row_spec, row_spec, row_spec, w_spec, b_spec, w_spec, b_spec],
        out_specs=[row_spec, row_spec],
        out_shape=[jax.ShapeDtypeStruct((N, D), jnp.float32),
                   jax.ShapeDtypeStruct((N, D), jnp.float32)],
    )(y_real, u_real, u_imag, Wr, br, Wi, bi)


def _pack_bf16_pair(hi_f32, lo_f32):
    def rnd(x):
        u = lax.bitcast_convert_type(x, jnp.uint32)
        u = (u + jnp.uint32(0x7FFF) + ((u >> jnp.uint32(16)) & jnp.uint32(1)))
        return u & jnp.uint32(0xFFFF0000)
    hi = rnd(hi_f32)
    lo = rnd(lo_f32) >> jnp.uint32(16)
    return lax.bitcast_convert_type(hi | lo, jnp.int32)


def _unblock(a):
    return a.reshape(NPASS * NW, N, FPW).transpose(1, 0, 2).reshape(N, D)


def kernel(x_real, x_imag, edge_index, Wr, br, Wi, bi):
    row, col = edge_index[0], edge_index[1]
    packed = _pack_bf16_pair(x_real, x_imag)
    xp = packed.reshape(N, NPASS * NW, FPW).transpose(1, 0, 2).reshape(
        NPASS * NW, FPW * N)
    yr_o, ur_o, ui_o = _sc_spmm(row, col, xp)
    y_real = _unblock(yr_o)
    u_real = _unblock(ur_o)
    u_imag = _unblock(ui_o)
    return _dense_pallas(y_real, u_real, u_imag, Wr, br, Wi, bi)


# speedup vs baseline: 6.8444x; 1.1829x over previous
import functools

import jax
import jax.numpy as jnp
from jax import lax
from jax.experimental import pallas as pl
from jax.experimental.pallas import tpu as pltpu
from jax.experimental.pallas import tpu_sc as plsc

N = 10000
D = 128
K = 3
ALPHA = 0.5

NW = 32
FPW = 2
NPASS = D // (NW * FPW)
NB = NPASS * NW
C = 1280
ROT = 7

_ROW_BLK = 2000

_MAGIC = 0x5F3759DF
_MASKHI = -65536


def _rsqrt_nr(x):
    u = plsc.bitcast(x, jnp.int32)
    u = jnp.int32(_MAGIC) - lax.shift_right_logical(u, jnp.int32(1))
    r = plsc.bitcast(u, jnp.float32)
    for _ in range(3):
        r = r * (1.5 - 0.5 * x * r * r)
    return r


def _unpack_pair(p):
    hi = plsc.bitcast(p & jnp.int32(_MASKHI), jnp.float32)
    lo = plsc.bitcast(lax.shift_left(p, jnp.int32(16)), jnp.float32)
    return hi, lo


def _sc_spmm(row, col, xp):
    E = row.shape[0]
    assert E % C == 0
    nchunk = E // C
    npair = nchunk // 2
    mesh = plsc.VectorSubcoreMesh(core_axis_name="c", subcore_axis_name="s")
    out_t = jax.ShapeDtypeStruct((NB, FPW, N), jnp.float32)

    @functools.partial(
        pl.kernel, mesh=mesh,
        out_type=[out_t, out_t, out_t],
        compiler_params=pltpu.CompilerParams(needs_layout_passes=False),
        scratch_types=[
            pltpu.VMEM((N,), jnp.float32),
            pltpu.VMEM((N,), jnp.float32),
            pltpu.VMEM((N,), jnp.int32),
            pltpu.VMEM((N,), jnp.int32),
            pltpu.VMEM((N,), jnp.float32),
            pltpu.VMEM((N,), jnp.float32),
            pltpu.VMEM((N,), jnp.float32),
            pltpu.VMEM((N,), jnp.float32),
            pltpu.VMEM((N,), jnp.float32),
            pltpu.VMEM((N,), jnp.float32),
            pltpu.VMEM((C,), jnp.int32),
            pltpu.VMEM((C,), jnp.int32),
            pltpu.VMEM((C,), jnp.int32),
            pltpu.VMEM((C,), jnp.int32),
            pltpu.SemaphoreType.DMA,
            pltpu.SemaphoreType.DMA,
            pltpu.SemaphoreType.DMA,
            pltpu.SemaphoreType.DMA,
        ],
    )
    def k(row_h, col_h, xp_h, yr_h, ur_h, ui_h,
          da, db, xv0, xv1, yr0, yr1, yrt0, yrt1, ui0, ui1,
          rb0, rb1, cb0, cb1, sr0, sr1, sc0, sc1):
        wid = lax.axis_index("s") * 2 + lax.axis_index("c")
        rbufs, cbufs = (rb0, rb1), (cb0, cb1)
        srs, scs = (sr0, sr1), (sc0, sc1)
        xvs = (xv0, xv1)
        yrs, yrts, uis = (yr0, yr1), (yrt0, yrt1), (ui0, ui1)

        def chunk_of(c):
            return lax.rem(c + wid * ROT, nchunk)

        def start_chunk(ci, par):
            off = chunk_of(ci) * C
            pltpu.make_async_copy(
                row_h.at[pl.ds(off, C)], rbufs[par], srs[par]).start()
            pltpu.make_async_copy(
                col_h.at[pl.ds(off, C)], cbufs[par], scs[par]).start()

        def wait_chunk(par):
            pltpu.make_async_copy(
                row_h.at[pl.ds(0, C)], rbufs[par], srs[par]).wait()
            pltpu.make_async_copy(
                col_h.at[pl.ds(0, C)], cbufs[par], scs[par]).wait()

        def edge_pass(process_group):
            def pair_body(j, _):
                for par in (0, 1):
                    c = j * 2 + par
                    start_chunk(c + 1, 1 - par)
                    wait_chunk(par)

                    def g_body(g, _):
                        i0 = g * 16
                        rows = rbufs[par][pl.ds(i0, 16)]
                        cols = cbufs[par][pl.ds(i0, 16)]
                        process_group(rows, cols)
                        return 0

                    lax.fori_loop(0, C // 16, g_body, 0)
                return 0
            lax.fori_loop(0, npair, pair_body, 0)

        def zero_ref(ref, n):
            z = jnp.zeros((16,), ref.dtype)

            def b(i, _):
                ref[pl.ds(i * 16, 16)] = z
                return 0
            lax.fori_loop(0, n // 16, b, 0)

        zero_ref(da, N)
        zero_ref(db, N)
        start_chunk(0, 0)
        ones = jnp.ones((16,), jnp.float32)

        def deg_group(rows, cols):
            plsc.addupdate_scatter(da, [rows], ones)
            plsc.addupdate_scatter(db, [cols], ones)

        edge_pass(deg_group)

        def finalize(ref):
            def b(i, _):
                s = pl.ds(i * 16, 16)
                d = ref[s]
                r1 = _rsqrt_nr(d)
                val = r1 * _rsqrt_nr(r1)
                ref[s] = jnp.where(d > 0.0, val, 0.0)
                return 0
            lax.fori_loop(0, N // 16, b, 0)

        finalize(da)
        finalize(db)

        for p in range(NPASS):
            b_idx = wid + NW * p
            for f in range(FPW):
                pltpu.sync_copy(xp_h.at[b_idx, f], xvs[f])
                zero_ref(yrs[f], N)
                zero_ref(yrts[f], N)
                zero_ref(uis[f], N)

            def acc_group(rows, cols):
                av = plsc.load_gather(da, [rows])
                bv = plsc.load_gather(db, [cols])
                w = av * bv
                hw = w * ALPHA
                for f in range(FPW):
                    pc = plsc.load_gather(xvs[f], [cols])
                    pr = plsc.load_gather(xvs[f], [rows])
                    xr_c, xi_c = _unpack_pair(pc)
                    xr_r, xi_r = _unpack_pair(pr)
                    plsc.addupdate_scatter(yrs[f], [rows], w * xr_c)
                    plsc.addupdate_scatter(yrts[f], [cols], w * xr_r)
                    plsc.addupdate_scatter(uis[f], [rows], hw * xi_c)
                    plsc.addupdate_scatter(uis[f], [cols], hw * xi_r)

            edge_pass(acc_group)

            for f in range(FPW):
                def ucomb(i, _):
                    s = pl.ds(i * 16, 16)
                    yrts[f][s] = ALPHA * yrs[f][s] + (1.0 - ALPHA) * yrts[f][s]
                    return 0
                lax.fori_loop(0, N // 16, ucomb, 0)

                pltpu.sync_copy(yrs[f], yr_h.at[b_idx, f])
                pltpu.sync_copy(yrts[f], ur_h.at[b_idx, f])
                pltpu.sync_copy(uis[f], ui_h.at[b_idx, f])

        wait_chunk(0)

    return k(row, col, xp)


def _dense_body(yr_ref, ur_ref, ui_ref, wr_ref, br_ref, wi_ref, bi_ref,
                or_ref, oi_ref):
    wr_eff = wr_ref[0] + 0.5 * wr_ref[1] + 0.25 * wr_ref[2]
    wi_eff = wi_ref[0] + 0.5 * wi_ref[1] + 0.25 * wi_ref[2]
    br_eff = br_ref[0] + 0.5 * br_ref[1] + 0.25 * br_ref[2]
    bi_eff = bi_ref[0] + 0.5 * bi_ref[1] + 0.25 * bi_ref[2]

    def mm(x, w):
        return jax.lax.dot_general(
            x, w, (((1,), (1,)), ((), ())),
            preferred_element_type=jnp.float32)

    yr = yr_ref[...]
    ur = ur_ref[...]
    ui = ui_ref[...]
    or_ref[...] = mm(ur, wr_eff) - mm(ui, wi_eff) + (br_eff - bi_eff)[None, :]
    oi_ref[...] = mm(yr, wi_eff) + mm(ui, wr_eff) + (br_eff + bi_eff)[None, :]


def _dense_pallas(y_real, u_real, u_imag, Wr, br, Wi, bi):
    grid = (N // _ROW_BLK,)
    row_spec = pl.BlockSpec((_ROW_BLK, D), lambda i: (i, 0))
    w_spec = pl.BlockSpec((K, D, D), lambda i: (0, 0, 0))
    b_spec = pl.BlockSpec((K, D), lambda i: (0, 0))
    return pl.pallas_call(
        _dense_body,
        grid=grid,
        in_specs=[row_spec, row_spec, row_spec, w_spec, b_spec, w_spec, b_spec],
        out_specs=[row_spec, row_spec],
        out_shape=[jax.ShapeDtypeStruct((N, D), jnp.float32),
                   jax.ShapeDtypeStruct((N, D), jnp.float32)],
    )(y_real, u_real, u_imag, Wr, br, Wi, bi)


def _pack_bf16_pair(hi_f32, lo_f32):
    def rnd(x):
        u = lax.bitcast_convert_type(x, jnp.uint32)
        u = (u + jnp.uint32(0x7FFF) + ((u >> jnp.uint32(16)) & jnp.uint32(1)))
        return u & jnp.uint32(0xFFFF0000)
    hi = rnd(hi_f32)
    lo = rnd(lo_f32) >> jnp.uint32(16)
    return lax.bitcast_convert_type(hi | lo, jnp.int32)


def _unblock(a):
    return a.transpose(2, 0, 1).reshape(N, D)


def kernel(x_real, x_imag, edge_index, Wr, br, Wi, bi):
    row, col = edge_index[0], edge_index[1]
    packed = _pack_bf16_pair(x_real, x_imag)
    xp = packed.reshape(N, NB, FPW).transpose(1, 2, 0)
    yr_o, ur_o, ui_o = _sc_spmm(row, col, xp)
    y_real = _unblock(yr_o)
    u_real = _unblock(ur_o)
    u_imag = _unblock(ui_o)
    return _dense_pallas(y_real, u_real, u_imag, Wr, br, Wi, bi)


# speedup vs baseline: 6.9051x; 1.0089x over previous
import functools

import jax
import jax.numpy as jnp
from jax import lax
from jax.experimental import pallas as pl
from jax.experimental.pallas import tpu as pltpu
from jax.experimental.pallas import tpu_sc as plsc

N = 10000
D = 128
K = 3
ALPHA = 0.5

NW = 32
FPW = 2
NPASS = D // (NW * FPW)
NB = NPASS * NW
C = 1280
ROT = 7
UNROLL = 2

_ROW_BLK = 2000

_MAGIC = 0x5F3759DF
_MASKHI = -65536


def _rsqrt_nr(x):
    u = plsc.bitcast(x, jnp.int32)
    u = jnp.int32(_MAGIC) - lax.shift_right_logical(u, jnp.int32(1))
    r = plsc.bitcast(u, jnp.float32)
    for _ in range(3):
        r = r * (1.5 - 0.5 * x * r * r)
    return r


def _unpack_pair(p):
    hi = plsc.bitcast(p & jnp.int32(_MASKHI), jnp.float32)
    lo = plsc.bitcast(lax.shift_left(p, jnp.int32(16)), jnp.float32)
    return hi, lo


def _sc_spmm(row, col, xp):
    E = row.shape[0]
    assert E % C == 0
    nchunk = E // C
    npair = nchunk // 2
    mesh = plsc.VectorSubcoreMesh(core_axis_name="c", subcore_axis_name="s")
    out_t = jax.ShapeDtypeStruct((NB, FPW, N), jnp.float32)

    @functools.partial(
        pl.kernel, mesh=mesh,
        out_type=[out_t, out_t, out_t],
        compiler_params=pltpu.CompilerParams(needs_layout_passes=False),
        scratch_types=[
            pltpu.VMEM((N,), jnp.float32),
            pltpu.VMEM((N,), jnp.float32),
            pltpu.VMEM((N,), jnp.int32),
            pltpu.VMEM((N,), jnp.int32),
            pltpu.VMEM((N,), jnp.float32),
            pltpu.VMEM((N,), jnp.float32),
            pltpu.VMEM((N,), jnp.float32),
            pltpu.VMEM((N,), jnp.float32),
            pltpu.VMEM((N,), jnp.float32),
            pltpu.VMEM((N,), jnp.float32),
            pltpu.VMEM((C,), jnp.int32),
            pltpu.VMEM((C,), jnp.int32),
            pltpu.VMEM((C,), jnp.int32),
            pltpu.VMEM((C,), jnp.int32),
            pltpu.SemaphoreType.DMA,
            pltpu.SemaphoreType.DMA,
            pltpu.SemaphoreType.DMA,
            pltpu.SemaphoreType.DMA,
        ],
    )
    def k(row_h, col_h, xp_h, yr_h, ur_h, ui_h,
          da, db, xv0, xv1, yr0, yr1, yrt0, yrt1, ui0, ui1,
          rb0, rb1, cb0, cb1, sr0, sr1, sc0, sc1):
        wid = lax.axis_index("s") * 2 + lax.axis_index("c")
        rbufs, cbufs = (rb0, rb1), (cb0, cb1)
        srs, scs = (sr0, sr1), (sc0, sc1)
        xvs = (xv0, xv1)
        yrs, yrts, uis = (yr0, yr1), (yrt0, yrt1), (ui0, ui1)

        def chunk_of(c):
            return lax.rem(c + wid * ROT, nchunk)

        def start_chunk(ci, par):
            off = chunk_of(ci) * C
            pltpu.make_async_copy(
                row_h.at[pl.ds(off, C)], rbufs[par], srs[par]).start()
            pltpu.make_async_copy(
                col_h.at[pl.ds(off, C)], cbufs[par], scs[par]).start()

        def wait_chunk(par):
            pltpu.make_async_copy(
                row_h.at[pl.ds(0, C)], rbufs[par], srs[par]).wait()
            pltpu.make_async_copy(
                col_h.at[pl.ds(0, C)], cbufs[par], scs[par]).wait()

        def edge_pass(process_group):
            def pair_body(j, _):
                for par in (0, 1):
                    c = j * 2 + par
                    start_chunk(c + 1, 1 - par)
                    wait_chunk(par)

                    def g_body(g, _):
                        for u in range(UNROLL):
                            i0 = g * (16 * UNROLL) + u * 16
                            rows = rbufs[par][pl.ds(i0, 16)]
                            cols = cbufs[par][pl.ds(i0, 16)]
                            process_group(rows, cols)
                        return 0

                    lax.fori_loop(0, C // (16 * UNROLL), g_body, 0)
                return 0
            lax.fori_loop(0, npair, pair_body, 0)

        def zero_ref(ref, n):
            z = jnp.zeros((16,), ref.dtype)

            def b(i, _):
                ref[pl.ds(i * 16, 16)] = z
                return 0
            lax.fori_loop(0, n // 16, b, 0)

        zero_ref(da, N)
        zero_ref(db, N)
        start_chunk(0, 0)
        ones = jnp.ones((16,), jnp.float32)

        def deg_group(rows, cols):
            plsc.addupdate_scatter(da, [rows], ones)
            plsc.addupdate_scatter(db, [cols], ones)

        edge_pass(deg_group)

        def finalize(ref):
            def b(i, _):
                s = pl.ds(i * 16, 16)
                d = ref[s]
                r1 = _rsqrt_nr(d)
                val = r1 * _rsqrt_nr(r1)
                ref[s] = jnp.where(d > 0.0, val, 0.0)
                return 0
            lax.fori_loop(0, N // 16, b, 0)

        finalize(da)
        finalize(db)

        for p in range(NPASS):
            b_idx = wid + NW * p
            for f in range(FPW):
                pltpu.sync_copy(xp_h.at[b_idx, f], xvs[f])
                zero_ref(yrs[f], N)
                zero_ref(yrts[f], N)
                zero_ref(uis[f], N)

            def acc_group(rows, cols):
                av = plsc.load_gather(da, [rows])
                bv = plsc.load_gather(db, [cols])
                w = av * bv
                hw = w * ALPHA
                for f in range(FPW):
                    pc = plsc.load_gather(xvs[f], [cols])
                    pr = plsc.load_gather(xvs[f], [rows])
                    xr_c, xi_c = _unpack_pair(pc)
                    xr_r, xi_r = _unpack_pair(pr)
                    plsc.addupdate_scatter(yrs[f], [rows], w * xr_c)
                    plsc.addupdate_scatter(yrts[f], [cols], w * xr_r)
                    plsc.addupdate_scatter(uis[f], [rows], hw * xi_c)
                    plsc.addupdate_scatter(uis[f], [cols], hw * xi_r)

            edge_pass(acc_group)

            for f in range(FPW):
                def ucomb(i, _):
                    s = pl.ds(i * 16, 16)
                    yrts[f][s] = ALPHA * yrs[f][s] + (1.0 - ALPHA) * yrts[f][s]
                    return 0
                lax.fori_loop(0, N // 16, ucomb, 0)

                pltpu.sync_copy(yrs[f], yr_h.at[b_idx, f])
                pltpu.sync_copy(yrts[f], ur_h.at[b_idx, f])
                pltpu.sync_copy(uis[f], ui_h.at[b_idx, f])

        wait_chunk(0)

    return k(row, col, xp)


def _dense_body(yr_ref, ur_ref, ui_ref, wr_ref, br_ref, wi_ref, bi_ref,
                or_ref, oi_ref):
    wr_eff = wr_ref[0] + 0.5 * wr_ref[1] + 0.25 * wr_ref[2]
    wi_eff = wi_ref[0] + 0.5 * wi_ref[1] + 0.25 * wi_ref[2]
    br_eff = br_ref[0] + 0.5 * br_ref[1] + 0.25 * br_ref[2]
    bi_eff = bi_ref[0] + 0.5 * bi_ref[1] + 0.25 * bi_ref[2]

    def mm(x, w):
        return jax.lax.dot_general(
            x, w, (((1,), (1,)), ((), ())),
            preferred_element_type=jnp.float32)

    yr = yr_ref[...]
    ur = ur_ref[...]
    ui = ui_ref[...]
    or_ref[...] = mm(ur, wr_eff) - mm(ui, wi_eff) + (br_eff - bi_eff)[None, :]
    oi_ref[...] = mm(yr, wi_eff) + mm(ui, wr_eff) + (br_eff + bi_eff)[None, :]


def _dense_pallas(y_real, u_real, u_imag, Wr, br, Wi, bi):
    grid = (N // _ROW_BLK,)
    row_spec = pl.BlockSpec((_ROW_BLK, D), lambda i: (i, 0))
    w_spec = pl.BlockSpec((K, D, D), lambda i: (0, 0, 0))
    b_spec = pl.BlockSpec((K, D), lambda i: (0, 0))
    return pl.pallas_call(
        _dense_body,
        grid=grid,
        in_specs=[row_spec, row_spec, row_spec, w_spec, b_spec, w_spec, b_spec],
        out_specs=[row_spec, row_spec],
        out_shape=[jax.ShapeDtypeStruct((N, D), jnp.float32),
                   jax.ShapeDtypeStruct((N, D), jnp.float32)],
    )(y_real, u_real, u_imag, Wr, br, Wi, bi)


def _pack_bf16_pair(hi_f32, lo_f32):
    def rnd(x):
        u = lax.bitcast_convert_type(x, jnp.uint32)
        u = (u + jnp.uint32(0x7FFF) + ((u >> jnp.uint32(16)) & jnp.uint32(1)))
        return u & jnp.uint32(0xFFFF0000)
    hi = rnd(hi_f32)
    lo = rnd(lo_f32) >> jnp.uint32(16)
    return lax.bitcast_convert_type(hi | lo, jnp.int32)


def _unblock(a):
    return a.transpose(2, 0, 1).reshape(N, D)


def kernel(x_real, x_imag, edge_index, Wr, br, Wi, bi):
    row, col = edge_index[0], edge_index[1]
    packed = _pack_bf16_pair(x_real, x_imag)
    xp = packed.reshape(N, NB, FPW).transpose(1, 2, 0)
    yr_o, ur_o, ui_o = _sc_spmm(row, col, xp)
    y_real = _unblock(yr_o)
    u_real = _unblock(ur_o)
    u_imag = _unblock(ui_o)
    return _dense_pallas(y_real, u_real, u_imag, Wr, br, Wi, bi)


# speedup vs baseline: 8.0577x; 1.1669x over previous
import functools

import jax
import jax.numpy as jnp
from jax import lax
from jax.experimental import pallas as pl
from jax.experimental.pallas import tpu as pltpu
from jax.experimental.pallas import tpu_sc as plsc

N = 10000
D = 128
K = 3
ALPHA = 0.5

NW = 32
FPW = 2
NPASS = D // (NW * FPW)
NB = NPASS * NW
C = 1280
ROT = 7
UNROLL = 2

_ROW_BLK = 2000

_MAGIC = 0x5F3759DF
_MASKHI = -65536


def _rsqrt_nr(x):
    u = plsc.bitcast(x, jnp.int32)
    u = jnp.int32(_MAGIC) - lax.shift_right_logical(u, jnp.int32(1))
    r = plsc.bitcast(u, jnp.float32)
    for _ in range(3):
        r = r * (1.5 - 0.5 * x * r * r)
    return r


def _unpack_pair(p):
    hi = plsc.bitcast(p & jnp.int32(_MASKHI), jnp.float32)
    lo = plsc.bitcast(lax.shift_left(p, jnp.int32(16)), jnp.float32)
    return hi, lo


def _sc_spmm(row, col, xp):
    E = row.shape[0]
    assert E % C == 0
    nchunk = E // C
    npair = nchunk // 2
    mesh = plsc.VectorSubcoreMesh(core_axis_name="c", subcore_axis_name="s")
    out_t = jax.ShapeDtypeStruct((NB, FPW, N), jnp.float32)

    @functools.partial(
        pl.kernel, mesh=mesh,
        out_type=[out_t, out_t, out_t],
        compiler_params=pltpu.CompilerParams(needs_layout_passes=False),
        scratch_types=[
            pltpu.VMEM((N,), jnp.float32),
            pltpu.VMEM((24,), jnp.float32),
            pltpu.VMEM((N,), jnp.float32),
            pltpu.VMEM((24,), jnp.float32),
            pltpu.VMEM((N,), jnp.int32),
            pltpu.VMEM((24,), jnp.float32),
            pltpu.VMEM((N,), jnp.int32),
            pltpu.VMEM((24,), jnp.float32),
            pltpu.VMEM((N,), jnp.float32),
            pltpu.VMEM((24,), jnp.float32),
            pltpu.VMEM((N,), jnp.float32),
            pltpu.VMEM((24,), jnp.float32),
            pltpu.VMEM((N,), jnp.float32),
            pltpu.VMEM((24,), jnp.float32),
            pltpu.VMEM((N,), jnp.float32),
            pltpu.VMEM((24,), jnp.float32),
            pltpu.VMEM((N,), jnp.float32),
            pltpu.VMEM((24,), jnp.float32),
            pltpu.VMEM((N,), jnp.float32),
            pltpu.VMEM((C,), jnp.int32),
            pltpu.VMEM((C,), jnp.int32),
            pltpu.VMEM((C,), jnp.int32),
            pltpu.VMEM((C,), jnp.int32),
            pltpu.SemaphoreType.DMA,
            pltpu.SemaphoreType.DMA,
            pltpu.SemaphoreType.DMA,
            pltpu.SemaphoreType.DMA,
        ],
    )
    def k(row_h, col_h, xp_h, yr_h, ur_h, ui_h,
          da, p0, db, p1, xv0, p2, xv1, p3, yr0, p4, yr1, p5,
          yrt0, p6, yrt1, p7, ui0, p8, ui1,
          rb0, rb1, cb0, cb1, sr0, sr1, sc0, sc1):
        wid = lax.axis_index("s") * 2 + lax.axis_index("c")
        zpad = jnp.zeros((16,), jnp.float32)
        for pr_ in (p0, p1, p2, p3, p4, p5, p6, p7, p8):
            pr_[pl.ds(0, 16)] = zpad
        rbufs, cbufs = (rb0, rb1), (cb0, cb1)
        srs, scs = (sr0, sr1), (sc0, sc1)
        xvs = (xv0, xv1)
        yrs, yrts, uis = (yr0, yr1), (yrt0, yrt1), (ui0, ui1)

        def chunk_of(c):
            return lax.rem(c + wid * ROT, nchunk)

        def start_chunk(ci, par):
            off = chunk_of(ci) * C
            pltpu.make_async_copy(
                row_h.at[pl.ds(off, C)], rbufs[par], srs[par]).start()
            pltpu.make_async_copy(
                col_h.at[pl.ds(off, C)], cbufs[par], scs[par]).start()

        def wait_chunk(par):
            pltpu.make_async_copy(
                row_h.at[pl.ds(0, C)], rbufs[par], srs[par]).wait()
            pltpu.make_async_copy(
                col_h.at[pl.ds(0, C)], cbufs[par], scs[par]).wait()

        def edge_pass(process_group):
            def pair_body(j, _):
                for par in (0, 1):
                    c = j * 2 + par
                    start_chunk(c + 1, 1 - par)
                    wait_chunk(par)

                    def g_body(g, _):
                        for u in range(UNROLL):
                            i0 = g * (16 * UNROLL) + u * 16
                            rows = rbufs[par][pl.ds(i0, 16)]
                            cols = cbufs[par][pl.ds(i0, 16)]
                            process_group(rows, cols)
                        return 0

                    lax.fori_loop(0, C // (16 * UNROLL), g_body, 0)
                return 0
            lax.fori_loop(0, npair, pair_body, 0)

        def zero_ref(ref, n):
            z = jnp.zeros((16,), ref.dtype)

            def b(i, _):
                ref[pl.ds(i * 16, 16)] = z
                return 0
            lax.fori_loop(0, n // 16, b, 0)

        zero_ref(da, N)
        zero_ref(db, N)
        start_chunk(0, 0)
        ones = jnp.ones((16,), jnp.float32)

        def deg_group(rows, cols):
            plsc.addupdate_scatter(da, [rows], ones)
            plsc.addupdate_scatter(db, [cols], ones)

        edge_pass(deg_group)

        def finalize(ref):
            def b(i, _):
                s = pl.ds(i * 16, 16)
                d = ref[s]
                r1 = _rsqrt_nr(d)
                val = r1 * _rsqrt_nr(r1)
                ref[s] = jnp.where(d > 0.0, val, 0.0)
                return 0
            lax.fori_loop(0, N // 16, b, 0)

        finalize(da)
        finalize(db)

        for p in range(NPASS):
            b_idx = wid + NW * p
            for f in range(FPW):
                pltpu.sync_copy(xp_h.at[b_idx, f], xvs[f])
                zero_ref(yrs[f], N)
                zero_ref(yrts[f], N)
                zero_ref(uis[f], N)

            def acc_group(rows, cols):
                av = plsc.load_gather(da, [rows])
                bv = plsc.load_gather(db, [cols])
                w = av * bv
                hw = w * ALPHA
                pc = [plsc.load_gather(xvs[f], [cols]) for f in range(FPW)]
                pr = [plsc.load_gather(xvs[f], [rows]) for f in range(FPW)]
                xc = [_unpack_pair(p) for p in pc]
                xr = [_unpack_pair(p) for p in pr]
                for f in range(FPW):
                    plsc.addupdate_scatter(yrs[f], [rows], w * xc[f][0])
                for f in range(FPW):
                    plsc.addupdate_scatter(yrts[f], [cols], w * xr[f][0])
                for f in range(FPW):
                    plsc.addupdate_scatter(uis[f], [rows], hw * xc[f][1])
                for f in range(FPW):
                    plsc.addupdate_scatter(uis[f], [cols], hw * xr[f][1])

            edge_pass(acc_group)

            for f in range(FPW):
                def ucomb(i, _):
                    s = pl.ds(i * 16, 16)
                    yrts[f][s] = ALPHA * yrs[f][s] + (1.0 - ALPHA) * yrts[f][s]
                    return 0
                lax.fori_loop(0, N // 16, ucomb, 0)

                pltpu.sync_copy(yrs[f], yr_h.at[b_idx, f])
                pltpu.sync_copy(yrts[f], ur_h.at[b_idx, f])
                pltpu.sync_copy(uis[f], ui_h.at[b_idx, f])

        wait_chunk(0)

    return k(row, col, xp)


def _dense_body(yr_ref, ur_ref, ui_ref, wr_ref, br_ref, wi_ref, bi_ref,
                or_ref, oi_ref):
    wr_eff = wr_ref[0] + 0.5 * wr_ref[1] + 0.25 * wr_ref[2]
    wi_eff = wi_ref[0] + 0.5 * wi_ref[1] + 0.25 * wi_ref[2]
    br_eff = br_ref[0] + 0.5 * br_ref[1] + 0.25 * br_ref[2]
    bi_eff = bi_ref[0] + 0.5 * bi_ref[1] + 0.25 * bi_ref[2]

    def mm(x, w):
        return jax.lax.dot_general(
            x, w, (((1,), (1,)), ((), ())),
            preferred_element_type=jnp.float32)

    yr = yr_ref[...]
    ur = ur_ref[...]
    ui = ui_ref[...]
    or_ref[...] = mm(ur, wr_eff) - mm(ui, wi_eff) + (br_eff - bi_eff)[None, :]
    oi_ref[...] = mm(yr, wi_eff) + mm(ui, wr_eff) + (br_eff + bi_eff)[None, :]


def _dense_pallas(y_real, u_real, u_imag, Wr, br, Wi, bi):
    grid = (N // _ROW_BLK,)
    row_spec = pl.BlockSpec((_ROW_BLK, D), lambda i: (i, 0))
    w_spec = pl.BlockSpec((K, D, D), lambda i: (0, 0, 0))
    b_spec = pl.BlockSpec((K, D), lambda i: (0, 0))
    return pl.pallas_call(
        _dense_body,
        grid=grid,
        in_specs=[row_spec, row_spec, row_spec, w_spec, b_spec, w_spec, b_spec],
        out_specs=[row_spec, row_spec],
        out_shape=[jax.ShapeDtypeStruct((N, D), jnp.float32),
                   jax.ShapeDtypeStruct((N, D), jnp.float32)],
    )(y_real, u_real, u_imag, Wr, br, Wi, bi)


def _pack_bf16_pair(hi_f32, lo_f32):
    def rnd(x):
        u = lax.bitcast_convert_type(x, jnp.uint32)
        u = (u + jnp.uint32(0x7FFF) + ((u >> jnp.uint32(16)) & jnp.uint32(1)))
        return u & jnp.uint32(0xFFFF0000)
    hi = rnd(hi_f32)
    lo = rnd(lo_f32) >> jnp.uint32(16)
    return lax.bitcast_convert_type(hi | lo, jnp.int32)


def _unblock(a):
    return a.transpose(2, 0, 1).reshape(N, D)


def kernel(x_real, x_imag, edge_index, Wr, br, Wi, bi):
    row, col = edge_index[0], edge_index[1]
    packed = _pack_bf16_pair(x_real, x_imag)
    xp = packed.reshape(N, NB, FPW).transpose(1, 2, 0)
    yr_o, ur_o, ui_o = _sc_spmm(row, col, xp)
    y_real = _unblock(yr_o)
    u_real = _unblock(ur_o)
    u_imag = _unblock(ui_o)
    return _dense_pallas(y_real, u_real, u_imag, Wr, br, Wi, bi)


# speedup vs baseline: 8.1135x; 1.0069x over previous
import functools

import jax
import jax.numpy as jnp
from jax import lax
from jax.experimental import pallas as pl
from jax.experimental.pallas import tpu as pltpu
from jax.experimental.pallas import tpu_sc as plsc

N = 10000
D = 128
K = 3
ALPHA = 0.5

NW = 32
FPW = 2
NPASS = D // (NW * FPW)
NB = NPASS * NW
C = 3200
ROT = 3
UNROLL = 2

_ROW_BLK = 2000

_MAGIC = 0x5F3759DF
_MASKHI = -65536


def _rsqrt_nr(x):
    u = plsc.bitcast(x, jnp.int32)
    u = jnp.int32(_MAGIC) - lax.shift_right_logical(u, jnp.int32(1))
    r = plsc.bitcast(u, jnp.float32)
    for _ in range(3):
        r = r * (1.5 - 0.5 * x * r * r)
    return r


def _unpack_pair(p):
    hi = plsc.bitcast(p & jnp.int32(_MASKHI), jnp.float32)
    lo = plsc.bitcast(lax.shift_left(p, jnp.int32(16)), jnp.float32)
    return hi, lo


def _sc_spmm(row, col, xp):
    E = row.shape[0]
    assert E % C == 0
    nchunk = E // C
    npair = nchunk // 2
    mesh = plsc.VectorSubcoreMesh(core_axis_name="c", subcore_axis_name="s")
    out_t = jax.ShapeDtypeStruct((NB, FPW, N), jnp.float32)

    @functools.partial(
        pl.kernel, mesh=mesh,
        out_type=[out_t, out_t, out_t],
        compiler_params=pltpu.CompilerParams(needs_layout_passes=False),
        scratch_types=[
            pltpu.VMEM((N,), jnp.float32),
            pltpu.VMEM((24,), jnp.float32),
            pltpu.VMEM((N,), jnp.float32),
            pltpu.VMEM((24,), jnp.float32),
            pltpu.VMEM((N,), jnp.int32),
            pltpu.VMEM((24,), jnp.float32),
            pltpu.VMEM((N,), jnp.int32),
            pltpu.VMEM((24,), jnp.float32),
            pltpu.VMEM((N,), jnp.float32),
            pltpu.VMEM((24,), jnp.float32),
            pltpu.VMEM((N,), jnp.float32),
            pltpu.VMEM((24,), jnp.float32),
            pltpu.VMEM((N,), jnp.float32),
            pltpu.VMEM((24,), jnp.float32),
            pltpu.VMEM((N,), jnp.float32),
            pltpu.VMEM((24,), jnp.float32),
            pltpu.VMEM((N,), jnp.float32),
            pltpu.VMEM((24,), jnp.float32),
            pltpu.VMEM((N,), jnp.float32),
            pltpu.VMEM((C,), jnp.int32),
            pltpu.VMEM((C,), jnp.int32),
            pltpu.VMEM((C,), jnp.int32),
            pltpu.VMEM((C,), jnp.int32),
            pltpu.SemaphoreType.DMA,
            pltpu.SemaphoreType.DMA,
            pltpu.SemaphoreType.DMA,
            pltpu.SemaphoreType.DMA,
        ],
    )
    def k(row_h, col_h, xp_h, yr_h, ur_h, ui_h,
          da, p0, db, p1, xv0, p2, xv1, p3, yr0, p4, yr1, p5,
          yrt0, p6, yrt1, p7, ui0, p8, ui1,
          rb0, rb1, cb0, cb1, sr0, sr1, sc0, sc1):
        wid = lax.axis_index("s") * 2 + lax.axis_index("c")
        zpad = jnp.zeros((16,), jnp.float32)
        for pr_ in (p0, p1, p2, p3, p4, p5, p6, p7, p8):
            pr_[pl.ds(0, 16)] = zpad
        rbufs, cbufs = (rb0, rb1), (cb0, cb1)
        srs, scs = (sr0, sr1), (sc0, sc1)
        xvs = (xv0, xv1)
        yrs, yrts, uis = (yr0, yr1), (yrt0, yrt1), (ui0, ui1)

        def chunk_of(c):
            return lax.rem(c + wid * ROT, nchunk)

        def start_chunk(ci, par):
            off = chunk_of(ci) * C
            pltpu.make_async_copy(
                row_h.at[pl.ds(off, C)], rbufs[par], srs[par]).start()
            pltpu.make_async_copy(
                col_h.at[pl.ds(off, C)], cbufs[par], scs[par]).start()

        def wait_chunk(par):
            pltpu.make_async_copy(
                row_h.at[pl.ds(0, C)], rbufs[par], srs[par]).wait()
            pltpu.make_async_copy(
                col_h.at[pl.ds(0, C)], cbufs[par], scs[par]).wait()

        def edge_pass(process_group):
            def pair_body(j, _):
                for par in (0, 1):
                    c = j * 2 + par
                    start_chunk(c + 1, 1 - par)
                    wait_chunk(par)

                    def g_body(g, _):
                        for u in range(UNROLL):
                            i0 = g * (16 * UNROLL) + u * 16
                            rows = rbufs[par][pl.ds(i0, 16)]
                            cols = cbufs[par][pl.ds(i0, 16)]
                            process_group(rows, cols)
                        return 0

                    lax.fori_loop(0, C // (16 * UNROLL), g_body, 0)
                return 0
            lax.fori_loop(0, npair, pair_body, 0)

        def zero_ref(ref, n):
            z = jnp.zeros((16,), ref.dtype)

            def b(i, _):
                ref[pl.ds(i * 16, 16)] = z
                return 0
            lax.fori_loop(0, n // 16, b, 0)

        zero_ref(da, N)
        zero_ref(db, N)
        start_chunk(0, 0)
        ones = jnp.ones((16,), jnp.float32)

        def deg_group(rows, cols):
            plsc.addupdate_scatter(da, [rows], ones)
            plsc.addupdate_scatter(db, [cols], ones)

        edge_pass(deg_group)

        def finalize(ref):
            def b(i, _):
                s = pl.ds(i * 16, 16)
                d = ref[s]
                r1 = _rsqrt_nr(d)
                val = r1 * _rsqrt_nr(r1)
                ref[s] = jnp.where(d > 0.0, val, 0.0)
                return 0
            lax.fori_loop(0, N // 16, b, 0)

        finalize(da)
        finalize(db)

        for p in range(NPASS):
            b_idx = wid + NW * p
            for f in range(FPW):
                pltpu.sync_copy(xp_h.at[b_idx, f], xvs[f])
                zero_ref(yrs[f], N)
                zero_ref(yrts[f], N)
                zero_ref(uis[f], N)

            def acc_group(rows, cols):
                av = plsc.load_gather(da, [rows])
                bv = plsc.load_gather(db, [cols])
                w = av * bv
                hw = w * ALPHA
                pc = [plsc.load_gather(xvs[f], [cols]) for f in range(FPW)]
                pr = [plsc.load_gather(xvs[f], [rows]) for f in range(FPW)]
                xc = [_unpack_pair(p) for p in pc]
                xr = [_unpack_pair(p) for p in pr]
                for f in range(FPW):
                    plsc.addupdate_scatter(yrs[f], [rows], w * xc[f][0])
                for f in range(FPW):
                    plsc.addupdate_scatter(yrts[f], [cols], w * xr[f][0])
                for f in range(FPW):
                    plsc.addupdate_scatter(uis[f], [rows], hw * xc[f][1])
                for f in range(FPW):
                    plsc.addupdate_scatter(uis[f], [cols], hw * xr[f][1])

            edge_pass(acc_group)

            for f in range(FPW):
                def ucomb(i, _):
                    s = pl.ds(i * 16, 16)
                    yrts[f][s] = ALPHA * yrs[f][s] + (1.0 - ALPHA) * yrts[f][s]
                    return 0
                lax.fori_loop(0, N // 16, ucomb, 0)

                pltpu.sync_copy(yrs[f], yr_h.at[b_idx, f])
                pltpu.sync_copy(yrts[f], ur_h.at[b_idx, f])
                pltpu.sync_copy(uis[f], ui_h.at[b_idx, f])

        wait_chunk(0)

    return k(row, col, xp)


def _dense_body(yr_ref, ur_ref, ui_ref, wr_ref, br_ref, wi_ref, bi_ref,
                or_ref, oi_ref):
    wr_eff = wr_ref[0] + 0.5 * wr_ref[1] + 0.25 * wr_ref[2]
    wi_eff = wi_ref[0] + 0.5 * wi_ref[1] + 0.25 * wi_ref[2]
    br_eff = br_ref[0] + 0.5 * br_ref[1] + 0.25 * br_ref[2]
    bi_eff = bi_ref[0] + 0.5 * bi_ref[1] + 0.25 * bi_ref[2]

    def mm(x, w):
        return jax.lax.dot_general(
            x, w, (((1,), (1,)), ((), ())),
            preferred_element_type=jnp.float32)

    yr = yr_ref[...]
    ur = ur_ref[...]
    ui = ui_ref[...]
    or_ref[...] = mm(ur, wr_eff) - mm(ui, wi_eff) + (br_eff - bi_eff)[None, :]
    oi_ref[...] = mm(yr, wi_eff) + mm(ui, wr_eff) + (br_eff + bi_eff)[None, :]


def _dense_pallas(y_real, u_real, u_imag, Wr, br, Wi, bi):
    grid = (N // _ROW_BLK,)
    row_spec = pl.BlockSpec((_ROW_BLK, D), lambda i: (i, 0))
    w_spec = pl.BlockSpec((K, D, D), lambda i: (0, 0, 0))
    b_spec = pl.BlockSpec((K, D), lambda i: (0, 0))
    return pl.pallas_call(
        _dense_body,
        grid=grid,
        in_specs=[row_spec, row_spec, row_spec, w_spec, b_spec, w_spec, b_spec],
        out_specs=[row_spec, row_spec],
        out_shape=[jax.ShapeDtypeStruct((N, D), jnp.float32),
                   jax.ShapeDtypeStruct((N, D), jnp.float32)],
    )(y_real, u_real, u_imag, Wr, br, Wi, bi)


def _pack_bf16_pair(hi_f32, lo_f32):
    def rnd(x):
        u = lax.bitcast_convert_type(x, jnp.uint32)
        u = (u + jnp.uint32(0x7FFF) + ((u >> jnp.uint32(16)) & jnp.uint32(1)))
        return u & jnp.uint32(0xFFFF0000)
    hi = rnd(hi_f32)
    lo = rnd(lo_f32) >> jnp.uint32(16)
    return lax.bitcast_convert_type(hi | lo, jnp.int32)


def _unblock(a):
    return a.transpose(2, 0, 1).reshape(N, D)


def kernel(x_real, x_imag, edge_index, Wr, br, Wi, bi):
    row, col = edge_index[0], edge_index[1]
    packed = _pack_bf16_pair(x_real, x_imag)
    xp = packed.reshape(N, NB, FPW).transpose(1, 2, 0)
    yr_o, ur_o, ui_o = _sc_spmm(row, col, xp)
    y_real = _unblock(yr_o)
    u_real = _unblock(ur_o)
    u_imag = _unblock(ui_o)
    return _dense_pallas(y_real, u_real, u_imag, Wr, br, Wi, bi)


# speedup vs baseline: 8.5329x; 1.0517x over previous
import functools

import jax
import jax.numpy as jnp
from jax import lax
from jax.experimental import pallas as pl
from jax.experimental.pallas import tpu as pltpu
from jax.experimental.pallas import tpu_sc as plsc

N = 10000
D = 128
K = 3
ALPHA = 0.5

NW = 32
FPW = 2
NPASS = D // (NW * FPW)
NB = NPASS * NW
C = 3200
ROT = 3
UNROLL = 2

_ROW_BLK = 2000

_MAGIC = 0x5F3759DF
_MASKHI = -65536


def _rsqrt_nr(x):
    u = plsc.bitcast(x, jnp.int32)
    u = jnp.int32(_MAGIC) - lax.shift_right_logical(u, jnp.int32(1))
    r = plsc.bitcast(u, jnp.float32)
    for _ in range(3):
        r = r * (1.5 - 0.5 * x * r * r)
    return r


def _unpack_pair(p):
    hi = plsc.bitcast(p & jnp.int32(_MASKHI), jnp.float32)
    lo = plsc.bitcast(lax.shift_left(p, jnp.int32(16)), jnp.float32)
    return hi, lo


def _sc_spmm(row, col, xp):
    E = row.shape[0]
    assert E % C == 0
    nchunk = E // C
    npair = nchunk // 2
    mesh = plsc.VectorSubcoreMesh(core_axis_name="c", subcore_axis_name="s")
    out_t = jax.ShapeDtypeStruct((NB, FPW, N), jnp.float32)

    @functools.partial(
        pl.kernel, mesh=mesh,
        out_type=[out_t, out_t, out_t,
                  jax.ShapeDtypeStruct((2, E), jnp.float32)],
        compiler_params=pltpu.CompilerParams(needs_layout_passes=False),
        scratch_types=[
            pltpu.VMEM((N,), jnp.float32),
            pltpu.VMEM((24,), jnp.float32),
            pltpu.VMEM((N,), jnp.float32),
            pltpu.VMEM((24,), jnp.float32),
            pltpu.VMEM((N,), jnp.int32),
            pltpu.VMEM((24,), jnp.float32),
            pltpu.VMEM((N,), jnp.int32),
            pltpu.VMEM((24,), jnp.float32),
            pltpu.VMEM((N,), jnp.float32),
            pltpu.VMEM((24,), jnp.float32),
            pltpu.VMEM((N,), jnp.float32),
            pltpu.VMEM((24,), jnp.float32),
            pltpu.VMEM((N,), jnp.float32),
            pltpu.VMEM((24,), jnp.float32),
            pltpu.VMEM((N,), jnp.float32),
            pltpu.VMEM((24,), jnp.float32),
            pltpu.VMEM((N,), jnp.float32),
            pltpu.VMEM((24,), jnp.float32),
            pltpu.VMEM((N,), jnp.float32),
            pltpu.VMEM((C,), jnp.int32),
            pltpu.VMEM((C,), jnp.int32),
            pltpu.VMEM((C,), jnp.int32),
            pltpu.VMEM((C,), jnp.int32),
            pltpu.VMEM((C,), jnp.float32),
            pltpu.VMEM((C,), jnp.float32),
            pltpu.SemaphoreType.DMA,
            pltpu.SemaphoreType.DMA,
            pltpu.SemaphoreType.DMA,
            pltpu.SemaphoreType.DMA,
            pltpu.SemaphoreType.DMA,
            pltpu.SemaphoreType.DMA,
        ],
    )
    def k(row_h, col_h, xp_h, yr_h, ur_h, ui_h, w_h,
          da, p0, db, p1, xv0, p2, xv1, p3, yr0, p4, yr1, p5,
          yrt0, p6, yrt1, p7, ui0, p8, ui1,
          rb0, rb1, cb0, cb1, wb0, wb1,
          sr0, sr1, sc0, sc1, sw0, sw1):
        sid = lax.axis_index("s")
        cid = lax.axis_index("c")
        wid = sid * 2 + cid
        zpad = jnp.zeros((16,), jnp.float32)
        for pr_ in (p0, p1, p2, p3, p4, p5, p6, p7, p8):
            pr_[pl.ds(0, 16)] = zpad
        rbufs, cbufs, wbufs = (rb0, rb1), (cb0, cb1), (wb0, wb1)
        srs, scs, sws = (sr0, sr1), (sc0, sc1), (sw0, sw1)
        xvs = (xv0, xv1)
        yrs, yrts, uis = (yr0, yr1), (yrt0, yrt1), (ui0, ui1)

        def chunk_of(c):
            return lax.rem(c + wid * ROT, nchunk)

        def start_chunk(ci, par, with_w=False):
            off = chunk_of(ci) * C
            pltpu.make_async_copy(
                row_h.at[pl.ds(off, C)], rbufs[par], srs[par]).start()
            pltpu.make_async_copy(
                col_h.at[pl.ds(off, C)], cbufs[par], scs[par]).start()
            if with_w:
                pltpu.make_async_copy(
                    w_h.at[cid, pl.ds(off, C)], wbufs[par], sws[par]).start()

        def wait_chunk(par, with_w=False):
            pltpu.make_async_copy(
                row_h.at[pl.ds(0, C)], rbufs[par], srs[par]).wait()
            pltpu.make_async_copy(
                col_h.at[pl.ds(0, C)], cbufs[par], scs[par]).wait()
            if with_w:
                pltpu.make_async_copy(
                    w_h.at[cid, pl.ds(0, C)], wbufs[par], sws[par]).wait()

        def edge_pass(process_group, with_w=False):
            def pair_body(j, _):
                for par in (0, 1):
                    c = j * 2 + par
                    start_chunk(c + 1, 1 - par, with_w)
                    wait_chunk(par, with_w)

                    def g_body(g, _):
                        for u in range(UNROLL):
                            i0 = g * (16 * UNROLL) + u * 16
                            rows = rbufs[par][pl.ds(i0, 16)]
                            cols = cbufs[par][pl.ds(i0, 16)]
                            process_group(par, i0, rows, cols)
                        return 0

                    lax.fori_loop(0, C // (16 * UNROLL), g_body, 0)
                return 0
            lax.fori_loop(0, npair, pair_body, 0)

        def zero_ref(ref, n):
            z = jnp.zeros((16,), ref.dtype)

            def b(i, _):
                ref[pl.ds(i * 16, 16)] = z
                return 0
            lax.fori_loop(0, n // 16, b, 0)

        zero_ref(da, N)
        zero_ref(db, N)
        start_chunk(0, 0)
        ones = jnp.ones((16,), jnp.float32)

        def deg_group(par, i0, rows, cols):
            plsc.addupdate_scatter(da, [rows], ones)
            plsc.addupdate_scatter(db, [cols], ones)

        edge_pass(deg_group)

        def finalize(ref):
            def b(i, _):
                s = pl.ds(i * 16, 16)
                d = ref[s]
                r1 = _rsqrt_nr(d)
                val = r1 * _rsqrt_nr(r1)
                ref[s] = jnp.where(d > 0.0, val, 0.0)
                return 0
            lax.fori_loop(0, N // 16, b, 0)

        finalize(da)
        finalize(db)

        for j in range((nchunk + 15) // 16):
            ci = j * 16 + sid

            @pl.when(ci < nchunk)
            def _():
                off = ci * C
                r_cp = pltpu.make_async_copy(
                    row_h.at[pl.ds(off, C)], rb1, sr1)
                c_cp = pltpu.make_async_copy(
                    col_h.at[pl.ds(off, C)], cb1, sc1)
                r_cp.start()
                c_cp.start()
                r_cp.wait()
                c_cp.wait()

                def wg(g, _):
                    i0 = g * 16
                    rows = rb1[pl.ds(i0, 16)]
                    cols = cb1[pl.ds(i0, 16)]
                    av = plsc.load_gather(da, [rows])
                    bv = plsc.load_gather(db, [cols])
                    wb1[pl.ds(i0, 16)] = av * bv
                    return 0

                lax.fori_loop(0, C // 16, wg, 0)
                pltpu.sync_copy(wb1, w_h.at[cid, pl.ds(off, C)])

        plsc.subcore_barrier()
        pltpu.make_async_copy(
            w_h.at[cid, pl.ds(chunk_of(0) * C, C)], wb0, sw0).start()

        for p in range(NPASS):
            b_idx = wid + NW * p
            for f in range(FPW):
                pltpu.sync_copy(xp_h.at[b_idx, f], xvs[f])
                zero_ref(yrs[f], N)
                zero_ref(yrts[f], N)
                zero_ref(uis[f], N)

            def acc_group(par, i0, rows, cols):
                w = wbufs[par][pl.ds(i0, 16)]
                hw = w * ALPHA
                pc = [plsc.load_gather(xvs[f], [cols]) for f in range(FPW)]
                pr = [plsc.load_gather(xvs[f], [rows]) for f in range(FPW)]
                xc = [_unpack_pair(p) for p in pc]
                xr = [_unpack_pair(p) for p in pr]
                for f in range(FPW):
                    plsc.addupdate_scatter(yrs[f], [rows], w * xc[f][0])
                for f in range(FPW):
                    plsc.addupdate_scatter(yrts[f], [cols], w * xr[f][0])
                for f in range(FPW):
                    plsc.addupdate_scatter(uis[f], [rows], hw * xc[f][1])
                for f in range(FPW):
                    plsc.addupdate_scatter(uis[f], [cols], hw * xr[f][1])

            edge_pass(acc_group, with_w=True)

            for f in range(FPW):
                def ucomb(i, _):
                    s = pl.ds(i * 16, 16)
                    yrts[f][s] = ALPHA * yrs[f][s] + (1.0 - ALPHA) * yrts[f][s]
                    return 0
                lax.fori_loop(0, N // 16, ucomb, 0)

                pltpu.sync_copy(yrs[f], yr_h.at[b_idx, f])
                pltpu.sync_copy(yrts[f], ur_h.at[b_idx, f])
                pltpu.sync_copy(uis[f], ui_h.at[b_idx, f])

        wait_chunk(0, with_w=True)

    yr_o, ur_o, ui_o, _unused_w = k(row, col, xp)
    return yr_o, ur_o, ui_o


def _dense_body(yr_ref, ur_ref, ui_ref, wr_ref, br_ref, wi_ref, bi_ref,
                or_ref, oi_ref):
    wr_eff = wr_ref[0] + 0.5 * wr_ref[1] + 0.25 * wr_ref[2]
    wi_eff = wi_ref[0] + 0.5 * wi_ref[1] + 0.25 * wi_ref[2]
    br_eff = br_ref[0] + 0.5 * br_ref[1] + 0.25 * br_ref[2]
    bi_eff = bi_ref[0] + 0.5 * bi_ref[1] + 0.25 * bi_ref[2]

    def mm(x, w):
        return jax.lax.dot_general(
            x, w, (((1,), (1,)), ((), ())),
            preferred_element_type=jnp.float32)

    yr = yr_ref[...]
    ur = ur_ref[...]
    ui = ui_ref[...]
    or_ref[...] = mm(ur, wr_eff) - mm(ui, wi_eff) + (br_eff - bi_eff)[None, :]
    oi_ref[...] = mm(yr, wi_eff) + mm(ui, wr_eff) + (br_eff + bi_eff)[None, :]


def _dense_pallas(y_real, u_real, u_imag, Wr, br, Wi, bi):
    grid = (N // _ROW_BLK,)
    row_spec = pl.BlockSpec((_ROW_BLK, D), lambda i: (i, 0))
    w_spec = pl.BlockSpec((K, D, D), lambda i: (0, 0, 0))
    b_spec = pl.BlockSpec((K, D), lambda i: (0, 0))
    return pl.pallas_call(
        _dense_body,
        grid=grid,
        in_specs=[row_spec, row_spec, row_spec, w_spec, b_spec, w_spec, b_spec],
        out_specs=[row_spec, row_spec],
        out_shape=[jax.ShapeDtypeStruct((N, D), jnp.float32),
                   jax.ShapeDtypeStruct((N, D), jnp.float32)],
    )(y_real, u_real, u_imag, Wr, br, Wi, bi)


def _pack_bf16_pair(hi_f32, lo_f32):
    def rnd(x):
        u = lax.bitcast_convert_type(x, jnp.uint32)
        u = (u + jnp.uint32(0x7FFF) + ((u >> jnp.uint32(16)) & jnp.uint32(1)))
        return u & jnp.uint32(0xFFFF0000)
    hi = rnd(hi_f32)
    lo = rnd(lo_f32) >> jnp.uint32(16)
    return lax.bitcast_convert_type(hi | lo, jnp.int32)


def _unblock(a):
    return a.transpose(2, 0, 1).reshape(N, D)


def kernel(x_real, x_imag, edge_index, Wr, br, Wi, bi):
    row, col = edge_index[0], edge_index[1]
    packed = _pack_bf16_pair(x_real, x_imag)
    xp = packed.reshape(N, NB, FPW).transpose(1, 2, 0)
    yr_o, ur_o, ui_o = _sc_spmm(row, col, xp)
    y_real = _unblock(yr_o)
    u_real = _unblock(ur_o)
    u_imag = _unblock(ui_o)
    return _dense_pallas(y_real, u_real, u_imag, Wr, br, Wi, bi)


# speedup vs baseline: 8.6483x; 1.0135x over previous
import functools

import jax
import jax.numpy as jnp
from jax import lax
from jax.experimental import pallas as pl
from jax.experimental.pallas import tpu as pltpu
from jax.experimental.pallas import tpu_sc as plsc

N = 10000
D = 128
K = 3
ALPHA = 0.5

NW = 32
FPW = 2
NPASS = D // (NW * FPW)
NB = NPASS * NW
C = 3200
ROT = 3
UNROLL = 2

_ROW_BLK = 2000

_MAGIC = 0x5F3759DF
_MASKHI = -65536


def _rsqrt_nr(x):
    u = plsc.bitcast(x, jnp.int32)
    u = jnp.int32(_MAGIC) - lax.shift_right_logical(u, jnp.int32(1))
    r = plsc.bitcast(u, jnp.float32)
    for _ in range(3):
        r = r * (1.5 - 0.5 * x * r * r)
    return r


def _unpack_pair(p):
    hi = plsc.bitcast(p & jnp.int32(_MASKHI), jnp.float32)
    lo = plsc.bitcast(lax.shift_left(p, jnp.int32(16)), jnp.float32)
    return hi, lo


def _sc_spmm(row, col, xp):
    E = row.shape[0]
    assert E % C == 0
    nchunk = E // C
    npair = nchunk // 2
    mesh = plsc.VectorSubcoreMesh(core_axis_name="c", subcore_axis_name="s")
    out_t = jax.ShapeDtypeStruct((NB, FPW, N), jnp.float32)

    @functools.partial(
        pl.kernel, mesh=mesh,
        out_type=[out_t, out_t, out_t,
                  jax.ShapeDtypeStruct((2, E), jnp.float32)],
        compiler_params=pltpu.CompilerParams(needs_layout_passes=False),
        scratch_types=[
            pltpu.VMEM((N,), jnp.float32),
            pltpu.VMEM((24,), jnp.float32),
            pltpu.VMEM((N,), jnp.float32),
            pltpu.VMEM((24,), jnp.float32),
            pltpu.VMEM((N,), jnp.int32),
            pltpu.VMEM((24,), jnp.float32),
            pltpu.VMEM((N,), jnp.int32),
            pltpu.VMEM((24,), jnp.float32),
            pltpu.VMEM((N,), jnp.float32),
            pltpu.VMEM((24,), jnp.float32),
            pltpu.VMEM((N,), jnp.float32),
            pltpu.VMEM((24,), jnp.float32),
            pltpu.VMEM((N,), jnp.float32),
            pltpu.VMEM((24,), jnp.float32),
            pltpu.VMEM((N,), jnp.float32),
            pltpu.VMEM((24,), jnp.float32),
            pltpu.VMEM((N,), jnp.float32),
            pltpu.VMEM((24,), jnp.float32),
            pltpu.VMEM((N,), jnp.float32),
            pltpu.VMEM((C,), jnp.int32),
            pltpu.VMEM((C,), jnp.int32),
            pltpu.VMEM((C,), jnp.int32),
            pltpu.VMEM((C,), jnp.int32),
            pltpu.VMEM((C,), jnp.float32),
            pltpu.VMEM((C,), jnp.float32),
            pltpu.SemaphoreType.DMA,
            pltpu.SemaphoreType.DMA,
            pltpu.SemaphoreType.DMA,
            pltpu.SemaphoreType.DMA,
            pltpu.SemaphoreType.DMA,
            pltpu.SemaphoreType.DMA,
        ],
    )
    def k(row_h, col_h, xp_h, yr_h, ur_h, ui_h, w_h,
          da, p0, db, p1, xv0, p2, xv1, p3, yr0, p4, yr1, p5,
          yrt0, p6, yrt1, p7, ui0, p8, ui1,
          rb0, rb1, cb0, cb1, wb0, wb1,
          sr0, sr1, sc0, sc1, sw0, sw1):
        sid = lax.axis_index("s")
        cid = lax.axis_index("c")
        wid = sid * 2 + cid
        zpad = jnp.zeros((16,), jnp.float32)
        for pr_ in (p0, p1, p2, p3, p4, p5, p6, p7, p8):
            pr_[pl.ds(0, 16)] = zpad
        rbufs, cbufs, wbufs = (rb0, rb1), (cb0, cb1), (wb0, wb1)
        srs, scs, sws = (sr0, sr1), (sc0, sc1), (sw0, sw1)
        xvs = (xv0, xv1)
        yrs, yrts, uis = (yr0, yr1), (yrt0, yrt1), (ui0, ui1)

        def chunk_of(c):
            return lax.rem(c + wid * ROT, nchunk)

        def start_chunk(ci, par, with_w=False):
            off = chunk_of(ci) * C
            pltpu.make_async_copy(
                row_h.at[pl.ds(off, C)], rbufs[par], srs[par]).start()
            pltpu.make_async_copy(
                col_h.at[pl.ds(off, C)], cbufs[par], scs[par]).start()
            if with_w:
                pltpu.make_async_copy(
                    w_h.at[cid, pl.ds(off, C)], wbufs[par], sws[par]).start()

        def wait_chunk(par, with_w=False):
            pltpu.make_async_copy(
                row_h.at[pl.ds(0, C)], rbufs[par], srs[par]).wait()
            pltpu.make_async_copy(
                col_h.at[pl.ds(0, C)], cbufs[par], scs[par]).wait()
            if with_w:
                pltpu.make_async_copy(
                    w_h.at[cid, pl.ds(0, C)], wbufs[par], sws[par]).wait()

        def edge_pass(process_group, with_w=False):
            def pair_body(j, _):
                for par in (0, 1):
                    c = j * 2 + par
                    start_chunk(c + 1, 1 - par, with_w)
                    wait_chunk(par, with_w)

                    def g_body(g, _):
                        for u in range(UNROLL):
                            i0 = g * (16 * UNROLL) + u * 16
                            rows = rbufs[par][pl.ds(i0, 16)]
                            cols = cbufs[par][pl.ds(i0, 16)]
                            process_group(par, i0, rows, cols)
                        return 0

                    lax.fori_loop(0, C // (16 * UNROLL), g_body, 0)
                return 0
            lax.fori_loop(0, npair, pair_body, 0)

        def zero_ref(ref, n):
            z = jnp.zeros((16,), ref.dtype)

            def b(i, _):
                ref[pl.ds(i * 16, 16)] = z
                return 0
            lax.fori_loop(0, n // 16, b, 0)

        zero_ref(da, N)
        zero_ref(db, N)
        start_chunk(0, 0)
        ones = jnp.ones((16,), jnp.float32)

        def deg_group(par, i0, rows, cols):
            plsc.addupdate_scatter(da, [rows], ones)
            plsc.addupdate_scatter(db, [cols], ones)

        edge_pass(deg_group)

        def finalize(ref):
            def b(i, _):
                s = pl.ds(i * 16, 16)
                d = ref[s]
                r1 = _rsqrt_nr(d)
                val = r1 * _rsqrt_nr(r1)
                ref[s] = jnp.where(d > 0.0, val, 0.0)
                return 0
            lax.fori_loop(0, N // 16, b, 0)

        finalize(da)
        finalize(db)

        for j in range((nchunk + 15) // 16):
            ci = j * 16 + sid

            @pl.when(ci < nchunk)
            def _():
                off = ci * C
                r_cp = pltpu.make_async_copy(
                    row_h.at[pl.ds(off, C)], rb1, sr1)
                c_cp = pltpu.make_async_copy(
                    col_h.at[pl.ds(off, C)], cb1, sc1)
                r_cp.start()
                c_cp.start()
                r_cp.wait()
                c_cp.wait()

                def wg(g, _):
                    i0 = g * 16
                    rows = rb1[pl.ds(i0, 16)]
                    cols = cb1[pl.ds(i0, 16)]
                    av = plsc.load_gather(da, [rows])
                    bv = plsc.load_gather(db, [cols])
                    wb1[pl.ds(i0, 16)] = av * bv
                    return 0

                lax.fori_loop(0, C // 16, wg, 0)
                pltpu.sync_copy(wb1, w_h.at[cid, pl.ds(off, C)])

        plsc.subcore_barrier()
        pltpu.make_async_copy(
            w_h.at[cid, pl.ds(chunk_of(0) * C, C)], wb0, sw0).start()

        for p in range(NPASS):
            b_idx = wid + NW * p
            for f in range(FPW):
                pltpu.sync_copy(xp_h.at[b_idx, f], xvs[f])
                zero_ref(yrs[f], N)
                zero_ref(yrts[f], N)
                zero_ref(uis[f], N)

            def acc_group(par, i0, rows, cols):
                w = wbufs[par][pl.ds(i0, 16)]
                hw = w * ALPHA
                pc = [plsc.load_gather(xvs[f], [cols]) for f in range(FPW)]
                pr = [plsc.load_gather(xvs[f], [rows]) for f in range(FPW)]
                xc = [_unpack_pair(p) for p in pc]
                xr = [_unpack_pair(p) for p in pr]
                for f in range(FPW):
                    plsc.addupdate_scatter(yrs[f], [rows], w * xc[f][0])
                for f in range(FPW):
                    plsc.addupdate_scatter(yrts[f], [cols], w * xr[f][0])
                for f in range(FPW):
                    plsc.addupdate_scatter(uis[f], [rows], hw * xc[f][1])
                for f in range(FPW):
                    plsc.addupdate_scatter(uis[f], [cols], hw * xr[f][1])

            edge_pass(acc_group, with_w=True)

            for f in range(FPW):
                pltpu.sync_copy(yrs[f], yr_h.at[b_idx, f])
                pltpu.sync_copy(yrts[f], ur_h.at[b_idx, f])
                pltpu.sync_copy(uis[f], ui_h.at[b_idx, f])

        wait_chunk(0, with_w=True)

    yr_o, ur_o, ui_o, _unused_w = k(row, col, xp)
    return yr_o, ur_o, ui_o


def _dense_body(yr_ref, yrt_ref, ui_ref, wr_ref, br_ref, wi_ref, bi_ref,
                or_ref, oi_ref):
    wr_eff = wr_ref[0] + 0.5 * wr_ref[1] + 0.25 * wr_ref[2]
    wi_eff = wi_ref[0] + 0.5 * wi_ref[1] + 0.25 * wi_ref[2]
    br_eff = br_ref[0] + 0.5 * br_ref[1] + 0.25 * br_ref[2]
    bi_eff = bi_ref[0] + 0.5 * bi_ref[1] + 0.25 * bi_ref[2]

    def mm(xb, w):
        return jax.lax.dot_general(
            xb, w, (((0,), (1,)), ((), ())),
            preferred_element_type=jnp.float32)

    yr = yr_ref[...]
    ui = ui_ref[...]
    ur = ALPHA * yr + (1.0 - ALPHA) * yrt_ref[...]
    or_ref[...] = mm(ur, wr_eff) - mm(ui, wi_eff) + (br_eff - bi_eff)[None, :]
    oi_ref[...] = mm(yr, wi_eff) + mm(ui, wr_eff) + (br_eff + bi_eff)[None, :]


def _dense_pallas(y_blk, yt_blk, ui_blk, Wr, br, Wi, bi):
    return pl.pallas_call(
        _dense_body,
        out_shape=[jax.ShapeDtypeStruct((N, D), jnp.float32),
                   jax.ShapeDtypeStruct((N, D), jnp.float32)],
    )(y_blk, yt_blk, ui_blk, Wr, br, Wi, bi)


def _pack_bf16_pair(hi_f32, lo_f32):
    def rnd(x):
        u = lax.bitcast_convert_type(x, jnp.uint32)
        u = (u + jnp.uint32(0x7FFF) + ((u >> jnp.uint32(16)) & jnp.uint32(1)))
        return u & jnp.uint32(0xFFFF0000)
    hi = rnd(hi_f32)
    lo = rnd(lo_f32) >> jnp.uint32(16)
    return lax.bitcast_convert_type(hi | lo, jnp.int32)


def kernel(x_real, x_imag, edge_index, Wr, br, Wi, bi):
    row, col = edge_index[0], edge_index[1]
    packed = _pack_bf16_pair(x_real, x_imag)
    xp = packed.reshape(N, NB, FPW).transpose(1, 2, 0)
    yr_o, yt_o, ui_o = _sc_spmm(row, col, xp)
    return _dense_pallas(yr_o.reshape(D, N), yt_o.reshape(D, N),
                         ui_o.reshape(D, N), Wr, br, Wi, bi)


# speedup vs baseline: 8.7147x; 1.0077x over previous
import functools

import jax
import jax.numpy as jnp
from jax import lax
from jax.experimental import pallas as pl
from jax.experimental.pallas import tpu as pltpu
from jax.experimental.pallas import tpu_sc as plsc

N = 10000
D = 128
K = 3
ALPHA = 0.5

NW = 32
FPW = 2
NPASS = D // (NW * FPW)
NB = NPASS * NW
C = 3200
ROT = 3
UNROLL = 4

_ROW_BLK = 2000

_MAGIC = 0x5F3759DF
_MASKHI = -65536


def _rsqrt_nr(x):
    u = plsc.bitcast(x, jnp.int32)
    u = jnp.int32(_MAGIC) - lax.shift_right_logical(u, jnp.int32(1))
    r = plsc.bitcast(u, jnp.float32)
    for _ in range(3):
        r = r * (1.5 - 0.5 * x * r * r)
    return r


def _unpack_pair(p):
    hi = plsc.bitcast(p & jnp.int32(_MASKHI), jnp.float32)
    lo = plsc.bitcast(lax.shift_left(p, jnp.int32(16)), jnp.float32)
    return hi, lo


def _sc_spmm(row, col, xp):
    E = row.shape[0]
    assert E % C == 0
    nchunk = E // C
    npair = nchunk // 2
    mesh = plsc.VectorSubcoreMesh(core_axis_name="c", subcore_axis_name="s")
    out_t = jax.ShapeDtypeStruct((NB, FPW, N), jnp.float32)

    @functools.partial(
        pl.kernel, mesh=mesh,
        out_type=[out_t, out_t, out_t,
                  jax.ShapeDtypeStruct((2, E), jnp.float32)],
        compiler_params=pltpu.CompilerParams(needs_layout_passes=False),
        scratch_types=[
            pltpu.VMEM((N,), jnp.float32),
            pltpu.VMEM((24,), jnp.float32),
            pltpu.VMEM((N,), jnp.float32),
            pltpu.VMEM((24,), jnp.float32),
            pltpu.VMEM((N,), jnp.int32),
            pltpu.VMEM((24,), jnp.float32),
            pltpu.VMEM((N,), jnp.int32),
            pltpu.VMEM((24,), jnp.float32),
            pltpu.VMEM((N,), jnp.float32),
            pltpu.VMEM((24,), jnp.float32),
            pltpu.VMEM((N,), jnp.float32),
            pltpu.VMEM((24,), jnp.float32),
            pltpu.VMEM((N,), jnp.float32),
            pltpu.VMEM((24,), jnp.float32),
            pltpu.VMEM((N,), jnp.float32),
            pltpu.VMEM((24,), jnp.float32),
            pltpu.VMEM((N,), jnp.float32),
            pltpu.VMEM((24,), jnp.float32),
            pltpu.VMEM((N,), jnp.float32),
            pltpu.VMEM((C,), jnp.int32),
            pltpu.VMEM((C,), jnp.int32),
            pltpu.VMEM((C,), jnp.int32),
            pltpu.VMEM((C,), jnp.int32),
            pltpu.VMEM((C,), jnp.float32),
            pltpu.VMEM((C,), jnp.float32),
            pltpu.SemaphoreType.DMA,
            pltpu.SemaphoreType.DMA,
            pltpu.SemaphoreType.DMA,
            pltpu.SemaphoreType.DMA,
            pltpu.SemaphoreType.DMA,
            pltpu.SemaphoreType.DMA,
        ],
    )
    def k(row_h, col_h, xp_h, yr_h, ur_h, ui_h, w_h,
          da, p0, db, p1, xv0, p2, xv1, p3, yr0, p4, yr1, p5,
          yrt0, p6, yrt1, p7, ui0, p8, ui1,
          rb0, rb1, cb0, cb1, wb0, wb1,
          sr0, sr1, sc0, sc1, sw0, sw1):
        sid = lax.axis_index("s")
        cid = lax.axis_index("c")
        wid = sid * 2 + cid
        zpad = jnp.zeros((16,), jnp.float32)
        for pr_ in (p0, p1, p2, p3, p4, p5, p6, p7, p8):
            pr_[pl.ds(0, 16)] = zpad
        rbufs, cbufs, wbufs = (rb0, rb1), (cb0, cb1), (wb0, wb1)
        srs, scs, sws = (sr0, sr1), (sc0, sc1), (sw0, sw1)
        xvs = (xv0, xv1)
        yrs, yrts, uis = (yr0, yr1), (yrt0, yrt1), (ui0, ui1)

        def chunk_of(c):
            return lax.rem(c + wid * ROT, nchunk)

        def start_chunk(ci, par, with_w=False):
            off = chunk_of(ci) * C
            pltpu.make_async_copy(
                row_h.at[pl.ds(off, C)], rbufs[par], srs[par]).start()
            pltpu.make_async_copy(
                col_h.at[pl.ds(off, C)], cbufs[par], scs[par]).start()
            if with_w:
                pltpu.make_async_copy(
                    w_h.at[cid, pl.ds(off, C)], wbufs[par], sws[par]).start()

        def wait_chunk(par, with_w=False):
            pltpu.make_async_copy(
                row_h.at[pl.ds(0, C)], rbufs[par], srs[par]).wait()
            pltpu.make_async_copy(
                col_h.at[pl.ds(0, C)], cbufs[par], scs[par]).wait()
            if with_w:
                pltpu.make_async_copy(
                    w_h.at[cid, pl.ds(0, C)], wbufs[par], sws[par]).wait()

        def edge_pass(process_group, with_w=False):
            def pair_body(j, _):
                for par in (0, 1):
                    c = j * 2 + par
                    start_chunk(c + 1, 1 - par, with_w)
                    wait_chunk(par, with_w)

                    def g_body(g, _):
                        for u in range(UNROLL):
                            i0 = g * (16 * UNROLL) + u * 16
                            rows = rbufs[par][pl.ds(i0, 16)]
                            cols = cbufs[par][pl.ds(i0, 16)]
                            process_group(par, i0, rows, cols)
                        return 0

                    lax.fori_loop(0, C // (16 * UNROLL), g_body, 0)
                return 0
            lax.fori_loop(0, npair, pair_body, 0)

        def zero_ref(ref, n):
            z = jnp.zeros((16,), ref.dtype)

            def b(i, _):
                ref[pl.ds(i * 16, 16)] = z
                return 0
            lax.fori_loop(0, n // 16, b, 0)

        zero_ref(da, N)
        zero_ref(db, N)
        start_chunk(0, 0)
        ones = jnp.ones((16,), jnp.float32)

        def deg_group(par, i0, rows, cols):
            plsc.addupdate_scatter(da, [rows], ones)
            plsc.addupdate_scatter(db, [cols], ones)

        edge_pass(deg_group)

        def finalize(ref):
            def b(i, _):
                s = pl.ds(i * 16, 16)
                d = ref[s]
                r1 = _rsqrt_nr(d)
                val = r1 * _rsqrt_nr(r1)
                ref[s] = jnp.where(d > 0.0, val, 0.0)
                return 0
            lax.fori_loop(0, N // 16, b, 0)

        finalize(da)
        finalize(db)

        for j in range((nchunk + 15) // 16):
            ci = j * 16 + sid

            @pl.when(ci < nchunk)
            def _():
                off = ci * C
                r_cp = pltpu.make_async_copy(
                    row_h.at[pl.ds(off, C)], rb1, sr1)
                c_cp = pltpu.make_async_copy(
                    col_h.at[pl.ds(off, C)], cb1, sc1)
                r_cp.start()
                c_cp.start()
                r_cp.wait()
                c_cp.wait()

                def wg(g, _):
                    i0 = g * 16
                    rows = rb1[pl.ds(i0, 16)]
                    cols = cb1[pl.ds(i0, 16)]
                    av = plsc.load_gather(da, [rows])
                    bv = plsc.load_gather(db, [cols])
                    wb1[pl.ds(i0, 16)] = av * bv
                    return 0

                lax.fori_loop(0, C // 16, wg, 0)
                pltpu.sync_copy(wb1, w_h.at[cid, pl.ds(off, C)])

        plsc.subcore_barrier()
        pltpu.make_async_copy(
            w_h.at[cid, pl.ds(chunk_of(0) * C, C)], wb0, sw0).start()

        for p in range(NPASS):
            b_idx = wid + NW * p
            for f in range(FPW):
                pltpu.sync_copy(xp_h.at[b_idx, f], xvs[f])
                zero_ref(yrs[f], N)
                zero_ref(yrts[f], N)
                zero_ref(uis[f], N)

            def acc_group(par, i0, rows, cols):
                w = wbufs[par][pl.ds(i0, 16)]
                hw = w * ALPHA
                pc = [plsc.load_gather(xvs[f], [cols]) for f in range(FPW)]
                pr = [plsc.load_gather(xvs[f], [rows]) for f in range(FPW)]
                xc = [_unpack_pair(p) for p in pc]
                xr = [_unpack_pair(p) for p in pr]
                for f in range(FPW):
                    plsc.addupdate_scatter(yrs[f], [rows], w * xc[f][0])
                for f in range(FPW):
                    plsc.addupdate_scatter(yrts[f], [cols], w * xr[f][0])
                for f in range(FPW):
                    plsc.addupdate_scatter(uis[f], [rows], hw * xc[f][1])
                for f in range(FPW):
                    plsc.addupdate_scatter(uis[f], [cols], hw * xr[f][1])

            edge_pass(acc_group, with_w=True)

            for f in range(FPW):
                pltpu.sync_copy(yrs[f], yr_h.at[b_idx, f])
                pltpu.sync_copy(yrts[f], ur_h.at[b_idx, f])
                pltpu.sync_copy(uis[f], ui_h.at[b_idx, f])

        wait_chunk(0, with_w=True)

    yr_o, ur_o, ui_o, _unused_w = k(row, col, xp)
    return yr_o, ur_o, ui_o


def _dense_body(yr_ref, yrt_ref, ui_ref, wr_ref, br_ref, wi_ref, bi_ref,
                or_ref, oi_ref):
    wr_eff = wr_ref[0] + 0.5 * wr_ref[1] + 0.25 * wr_ref[2]
    wi_eff = wi_ref[0] + 0.5 * wi_ref[1] + 0.25 * wi_ref[2]
    br_eff = br_ref[0] + 0.5 * br_ref[1] + 0.25 * br_ref[2]
    bi_eff = bi_ref[0] + 0.5 * bi_ref[1] + 0.25 * bi_ref[2]

    def mm(xb, w):
        return jax.lax.dot_general(
            xb, w, (((0,), (1,)), ((), ())),
            preferred_element_type=jnp.float32)

    yr = yr_ref[...]
    ui = ui_ref[...]
    ur = ALPHA * yr + (1.0 - ALPHA) * yrt_ref[...]
    or_ref[...] = mm(ur, wr_eff) - mm(ui, wi_eff) + (br_eff - bi_eff)[None, :]
    oi_ref[...] = mm(yr, wi_eff) + mm(ui, wr_eff) + (br_eff + bi_eff)[None, :]


def _dense_pallas(y_blk, yt_blk, ui_blk, Wr, br, Wi, bi):
    return pl.pallas_call(
        _dense_body,
        out_shape=[jax.ShapeDtypeStruct((N, D), jnp.float32),
                   jax.ShapeDtypeStruct((N, D), jnp.float32)],
    )(y_blk, yt_blk, ui_blk, Wr, br, Wi, bi)


def _pack_bf16_pair(hi_f32, lo_f32):
    def rnd(x):
        u = lax.bitcast_convert_type(x, jnp.uint32)
        u = (u + jnp.uint32(0x7FFF) + ((u >> jnp.uint32(16)) & jnp.uint32(1)))
        return u & jnp.uint32(0xFFFF0000)
    hi = rnd(hi_f32)
    lo = rnd(lo_f32) >> jnp.uint32(16)
    return lax.bitcast_convert_type(hi | lo, jnp.int32)


def kernel(x_real, x_imag, edge_index, Wr, br, Wi, bi):
    row, col = edge_index[0], edge_index[1]
    packed = _pack_bf16_pair(x_real, x_imag)
    xp = packed.reshape(N, NB, FPW).transpose(1, 2, 0)
    yr_o, yt_o, ui_o = _sc_spmm(row, col, xp)
    return _dense_pallas(yr_o.reshape(D, N), yt_o.reshape(D, N),
                         ui_o.reshape(D, N), Wr, br, Wi, bi)
